# baseline (speedup 1.0000x reference)
import jax
import jax.numpy as jnp
from jax.experimental import pallas as pl
from jax.experimental.pallas import tpu as pltpu

_BM = 200
_NBUF = 3
_NSLICE = 5
_FCHUNK = 400


def _issue_block(adj_hbm, abuf, sem, buf, base_row):
    rows = _BM // _NSLICE
    for s in range(_NSLICE):
        pltpu.make_async_copy(
            adj_hbm.at[pl.ds(base_row + s * rows, rows), :],
            abuf.at[buf, pl.ds(s * rows, rows), :],
            sem.at[buf],
        ).start()


def _wait_block(adj_hbm, abuf, sem, buf, base_row):
    rows = _BM // _NSLICE
    for s in range(_NSLICE):
        pltpu.make_async_copy(
            adj_hbm.at[pl.ds(base_row + s * rows, rows), :],
            abuf.at[buf, pl.ds(s * rows, rows), :],
            sem.at[buf],
        ).wait()


def _gcn_kernel(feat_hbm, w_ref, adj_hbm, out_ref,
                abuf, fw_ref, fbuf, adj_sem, f_sem):
    i = pl.program_id(0)
    nsteps = pl.num_programs(0)
    n = fw_ref.shape[0]

    @pl.when(i == 0)
    def _():
        _issue_block(adj_hbm, abuf, adj_sem, 0, 0)

        nchunks = n // _FCHUNK

        def fcopy(j):
            return pltpu.make_async_copy(
                feat_hbm.at[pl.ds(j * _FCHUNK, _FCHUNK), :],
                fbuf.at[j % 2],
                f_sem.at[j % 2],
            )

        fcopy(0).start()

        def body(j, carry):
            @pl.when(j + 1 < nchunks)
            def _():
                fcopy(j + 1).start()

            fcopy(j).wait()
            fw_ref[pl.ds(j * _FCHUNK, _FCHUNK), :] = jnp.dot(
                fbuf[j % 2], w_ref[...],
                preferred_element_type=jnp.float32).astype(jnp.bfloat16)
            return carry

        jax.lax.fori_loop(0, nchunks, body, 0)

        _issue_block(adj_hbm, abuf, adj_sem, 1 % _NBUF, _BM)

    @pl.when(jnp.logical_and(i + 2 < nsteps, True))
    def _():
        _issue_block(adj_hbm, abuf, adj_sem, (i + 2) % _NBUF, (i + 2) * _BM)

    _wait_block(adj_hbm, abuf, adj_sem, i % _NBUF, i * _BM)
    out_ref[...] = jax.lax.dot_general(
        abuf[i % _NBUF], fw_ref[...],
        dimension_numbers=(((1,), (0,)), ((), ())),
        precision=jax.lax.Precision.DEFAULT,
        preferred_element_type=jnp.float32)


def kernel(features, adj, weight):
    n, d_in = features.shape
    d_out = weight.shape[1]
    return pl.pallas_call(
        _gcn_kernel,
        grid=(pl.cdiv(n, _BM),),
        in_specs=[
            pl.BlockSpec(memory_space=pltpu.MemorySpace.HBM),
            pl.BlockSpec((d_in, d_out), lambda i: (0, 0)),
            pl.BlockSpec(memory_space=pltpu.MemorySpace.HBM),
        ],
        out_specs=pl.BlockSpec((_BM, d_out), lambda i: (i, 0)),
        out_shape=jax.ShapeDtypeStruct((n, d_out), jnp.float32),
        scratch_shapes=[
            pltpu.VMEM((_NBUF, _BM, n), jnp.float32),
            pltpu.VMEM((n, d_out), jnp.bfloat16),
            pltpu.VMEM((2, _FCHUNK, d_in), jnp.float32),
            pltpu.SemaphoreType.DMA((_NBUF,)),
            pltpu.SemaphoreType.DMA((2,)),
        ],
        compiler_params=pltpu.CompilerParams(
            dimension_semantics=("arbitrary",)),
    )(features, weight, adj)


# speedup vs baseline: 1.0781x; 1.0781x over previous
import jax
import jax.numpy as jnp
from jax.experimental import pallas as pl
from jax.experimental.pallas import tpu as pltpu

_BM = 200
_NBUF = 3
_NSLICE = 5
_FCHUNK = 2000


def _issue_block(adj_hbm, abuf, sem, buf, base_row):
    rows = _BM // _NSLICE
    for s in range(_NSLICE):
        pltpu.make_async_copy(
            adj_hbm.at[pl.ds(base_row + s * rows, rows), :],
            abuf.at[buf, pl.ds(s * rows, rows), :],
            sem.at[buf],
        ).start()


def _wait_block(adj_hbm, abuf, sem, buf, base_row):
    rows = _BM // _NSLICE
    for s in range(_NSLICE):
        pltpu.make_async_copy(
            adj_hbm.at[pl.ds(base_row + s * rows, rows), :],
            abuf.at[buf, pl.ds(s * rows, rows), :],
            sem.at[buf],
        ).wait()


def _gcn_kernel(feat_hbm, w_ref, adj_hbm, out_ref,
                abuf, fw_ref, fbuf, adj_sem, f_sem):
    i = pl.program_id(0)
    nsteps = pl.num_programs(0)
    n = fw_ref.shape[0]

    @pl.when(i == 0)
    def _():
        _issue_block(adj_hbm, abuf, adj_sem, 0, 0)

        nchunks = n // _FCHUNK

        def fcopy(j):
            return pltpu.make_async_copy(
                feat_hbm.at[pl.ds(j * _FCHUNK, _FCHUNK), :],
                fbuf.at[j % 2],
                f_sem.at[j % 2],
            )

        fcopy(0).start()

        def body(j, carry):
            @pl.when(j + 1 < nchunks)
            def _():
                fcopy(j + 1).start()

            fcopy(j).wait()
            fw_ref[pl.ds(j * _FCHUNK, _FCHUNK), :] = jnp.dot(
                fbuf[j % 2], w_ref[...],
                preferred_element_type=jnp.float32).astype(jnp.bfloat16)
            return carry

        jax.lax.fori_loop(0, nchunks, body, 0)

        _issue_block(adj_hbm, abuf, adj_sem, 1 % _NBUF, _BM)

    @pl.when(jnp.logical_and(i + 2 < nsteps, True))
    def _():
        _issue_block(adj_hbm, abuf, adj_sem, (i + 2) % _NBUF, (i + 2) * _BM)

    _wait_block(adj_hbm, abuf, adj_sem, i % _NBUF, i * _BM)
    out_ref[...] = jax.lax.dot_general(
        abuf[i % _NBUF], fw_ref[...],
        dimension_numbers=(((1,), (0,)), ((), ())),
        precision=jax.lax.Precision.DEFAULT,
        preferred_element_type=jnp.float32)


def kernel(features, adj, weight):
    n, d_in = features.shape
    d_out = weight.shape[1]
    return pl.pallas_call(
        _gcn_kernel,
        grid=(pl.cdiv(n, _BM),),
        in_specs=[
            pl.BlockSpec(memory_space=pltpu.MemorySpace.HBM),
            pl.BlockSpec((d_in, d_out), lambda i: (0, 0)),
            pl.BlockSpec(memory_space=pltpu.MemorySpace.HBM),
        ],
        out_specs=pl.BlockSpec((_BM, d_out), lambda i: (i, 0)),
        out_shape=jax.ShapeDtypeStruct((n, d_out), jnp.float32),
        scratch_shapes=[
            pltpu.VMEM((_NBUF, _BM, n), jnp.float32),
            pltpu.VMEM((n, d_out), jnp.bfloat16),
            pltpu.VMEM((2, _FCHUNK, d_in), jnp.float32),
            pltpu.SemaphoreType.DMA((_NBUF,)),
            pltpu.SemaphoreType.DMA((2,)),
        ],
        compiler_params=pltpu.CompilerParams(
            dimension_semantics=("arbitrary",)),
    )(features, weight, adj)
